# baseline (speedup 1.0000x reference)
import functools

import jax
import jax.numpy as jnp
from jax import lax
from jax.experimental import pallas as pl
from jax.experimental.pallas import tpu as pltpu
from jax.experimental.pallas import tpu_sc as plsc

_F = 384
_C = 384
_K_SIZE = _F * _C
_PB = 1792

_NW = 32
_CHUNK_ROWS = 4
_LANES = 128
_NNZ_PAD = _NW * _CHUNK_ROWS * _LANES
_SLICE = _K_SIZE // 16


def _scatter_body(idx_hbm, val_hbm, zero_hbm, out_hbm, idx_v, val_v, kacc):
    cid = lax.axis_index("c")
    sid = lax.axis_index("s")
    g = sid * 2 + cid
    pltpu.sync_copy(zero_hbm.at[pl.ds(sid * _SLICE, _SLICE)],
                    kacc.at[pl.ds(sid * _SLICE, _SLICE)])
    pltpu.sync_copy(idx_hbm.at[g], idx_v)
    pltpu.sync_copy(val_hbm.at[g], val_v)
    plsc.subcore_barrier()
    for j in range(_CHUNK_ROWS):
        pltpu.sync_copy(val_v.at[j], kacc.at[idx_v.at[j]], add=True)
    plsc.subcore_barrier()
    pltpu.sync_copy(kacc.at[pl.ds(sid * _SLICE, _SLICE)],
                    out_hbm.at[cid, pl.ds(sid * _SLICE, _SLICE)])


@jax.jit
def _build_kernel_coo(values, row_ids, col_ids):
    flat_idx = row_ids * _C + col_ids
    pad = _NNZ_PAD - values.shape[0]
    idx = jnp.concatenate([flat_idx, jnp.zeros((pad,), jnp.int32)])
    val = jnp.concatenate([values, jnp.zeros((pad,), jnp.float32)])
    idx = idx.reshape(_NW, _CHUNK_ROWS, _LANES)
    val = val.reshape(_NW, _CHUNK_ROWS, _LANES)
    zero = jnp.zeros((_K_SIZE,), jnp.float32)
    mesh = plsc.VectorSubcoreMesh(core_axis_name="c", subcore_axis_name="s")
    fn = functools.partial(
        pl.kernel,
        mesh=mesh,
        out_type=jax.ShapeDtypeStruct((2, _K_SIZE), jnp.float32),
        scratch_types=[
            pltpu.VMEM((_CHUNK_ROWS, _LANES), jnp.int32),
            pltpu.VMEM((_CHUNK_ROWS, _LANES), jnp.float32),
            pltpu.VMEM_SHARED((_K_SIZE,), jnp.float32),
        ],
    )(_scatter_body)
    return fn(idx, val, zero)


def _mm_body(k_ref, x_ref, o_ref):
    kmat = (k_ref[0] + k_ref[1]).astype(jnp.bfloat16)
    o_ref[...] = jax.lax.dot_general(
        kmat, x_ref[...].astype(jnp.bfloat16),
        dimension_numbers=(((1,), (0,)), ((), ())),
        preferred_element_type=jnp.float32,
    )


@functools.partial(jax.jit, static_argnames=("pb",))
def _matmul(kparts, x, pb=_PB):
    p = x.shape[1]
    return pl.pallas_call(
        _mm_body,
        grid=(p // pb,),
        in_specs=[
            pl.BlockSpec((2, _F, _C), lambda i: (0, 0, 0)),
            pl.BlockSpec((_C, pb), lambda i: (0, i)),
        ],
        out_specs=pl.BlockSpec((_F, pb), lambda i: (0, i)),
        out_shape=jax.ShapeDtypeStruct((_F, p), jnp.float32),
        compiler_params=pltpu.CompilerParams(
            dimension_semantics=("parallel",),
            allow_input_fusion=[False, True],
        ),
    )(kparts, x)


def kernel(inputs, values, row_ids, col_ids):
    b, c, h, w = inputs.shape
    kparts = _build_kernel_coo(values, row_ids, col_ids).reshape(2, _F, _C)
    flat = inputs.reshape(c, h * w) * jnp.float32(1.0)
    out = _matmul(kparts, flat)
    return out.reshape(b, _F, h, w)


# speedup vs baseline: 1.0839x; 1.0839x over previous
import functools

import jax
import jax.numpy as jnp
from jax import lax
from jax.experimental import pallas as pl
from jax.experimental.pallas import tpu as pltpu
from jax.experimental.pallas import tpu_sc as plsc

_F = 384
_C = 384
_K_SIZE = _F * _C
_PB = 3584

_NW = 32
_CHUNK_ROWS = 4
_LANES = 128
_NNZ_PAD = _NW * _CHUNK_ROWS * _LANES
_SLICE = _K_SIZE // 16


def _scatter_body(idx_hbm, val_hbm, zero_hbm, out_hbm, idx_v, val_v, kacc):
    cid = lax.axis_index("c")
    sid = lax.axis_index("s")
    g = sid * 2 + cid
    pltpu.sync_copy(zero_hbm.at[pl.ds(sid * _SLICE, _SLICE)],
                    kacc.at[pl.ds(sid * _SLICE, _SLICE)])
    pltpu.sync_copy(idx_hbm.at[g], idx_v)
    pltpu.sync_copy(val_hbm.at[g], val_v)
    plsc.subcore_barrier()
    for j in range(_CHUNK_ROWS):
        pltpu.sync_copy(val_v.at[j], kacc.at[idx_v.at[j]], add=True)
    plsc.subcore_barrier()
    pltpu.sync_copy(kacc.at[pl.ds(sid * _SLICE, _SLICE)],
                    out_hbm.at[cid, pl.ds(sid * _SLICE, _SLICE)])


@jax.jit
def _build_kernel_coo(values, row_ids, col_ids):
    flat_idx = row_ids * _C + col_ids
    pad = _NNZ_PAD - values.shape[0]
    idx = jnp.concatenate([flat_idx, jnp.zeros((pad,), jnp.int32)])
    val = jnp.concatenate([values, jnp.zeros((pad,), jnp.float32)])
    idx = idx.reshape(_NW, _CHUNK_ROWS, _LANES)
    val = val.reshape(_NW, _CHUNK_ROWS, _LANES)
    zero = jnp.zeros((_K_SIZE,), jnp.float32)
    mesh = plsc.VectorSubcoreMesh(core_axis_name="c", subcore_axis_name="s")
    fn = functools.partial(
        pl.kernel,
        mesh=mesh,
        out_type=jax.ShapeDtypeStruct((2, _K_SIZE), jnp.float32),
        scratch_types=[
            pltpu.VMEM((_CHUNK_ROWS, _LANES), jnp.int32),
            pltpu.VMEM((_CHUNK_ROWS, _LANES), jnp.float32),
            pltpu.VMEM_SHARED((_K_SIZE,), jnp.float32),
        ],
    )(_scatter_body)
    return fn(idx, val, zero)


def _mm_body(k_ref, x_ref, o_ref):
    kmat = (k_ref[0] + k_ref[1]).astype(jnp.bfloat16)
    o_ref[...] = jax.lax.dot_general(
        kmat, x_ref[...],
        dimension_numbers=(((1,), (0,)), ((), ())),
        preferred_element_type=jnp.float32,
    ).astype(jnp.bfloat16)


@functools.partial(jax.jit, static_argnames=("pb",))
def _matmul(kparts, x, pb=_PB):
    p = x.shape[1]
    return pl.pallas_call(
        _mm_body,
        grid=(p // pb,),
        in_specs=[
            pl.BlockSpec((2, _F, _C), lambda i: (0, 0, 0)),
            pl.BlockSpec((_C, pb), lambda i: (0, i)),
        ],
        out_specs=pl.BlockSpec((_F, pb), lambda i: (0, i)),
        out_shape=jax.ShapeDtypeStruct((_F, p), jnp.bfloat16),
        compiler_params=pltpu.CompilerParams(
            dimension_semantics=("parallel",),
        ),
    )(kparts, x)


def kernel(inputs, values, row_ids, col_ids):
    b, c, h, w = inputs.shape
    kparts = _build_kernel_coo(values, row_ids, col_ids).reshape(2, _F, _C)
    flat = inputs.reshape(c, h * w).astype(jnp.bfloat16)
    out = _matmul(kparts, flat)
    return out.astype(jnp.float32).reshape(b, _F, h, w)


# speedup vs baseline: 1.1000x; 1.0148x over previous
import functools

import jax
import jax.numpy as jnp
from jax import lax
from jax.experimental import pallas as pl
from jax.experimental.pallas import tpu as pltpu
from jax.experimental.pallas import tpu_sc as plsc

_F = 384
_C = 384
_K_SIZE = _F * _C
_PB = 7168

_NW = 32
_CHUNK_ROWS = 4
_LANES = 128
_NNZ_PAD = _NW * _CHUNK_ROWS * _LANES
_SLICE = _K_SIZE // 16


def _scatter_body(idx_hbm, val_hbm, zero_hbm, out_hbm, idx_v, val_v, kacc):
    cid = lax.axis_index("c")
    sid = lax.axis_index("s")
    g = sid * 2 + cid
    pltpu.sync_copy(zero_hbm.at[pl.ds(sid * _SLICE, _SLICE)],
                    kacc.at[pl.ds(sid * _SLICE, _SLICE)])
    pltpu.sync_copy(idx_hbm.at[g], idx_v)
    pltpu.sync_copy(val_hbm.at[g], val_v)
    plsc.subcore_barrier()
    for j in range(_CHUNK_ROWS):
        pltpu.sync_copy(val_v.at[j], kacc.at[idx_v.at[j]], add=True)
    plsc.subcore_barrier()
    pltpu.sync_copy(kacc.at[pl.ds(sid * _SLICE, _SLICE)],
                    out_hbm.at[cid, pl.ds(sid * _SLICE, _SLICE)])


@jax.jit
def _build_kernel_coo(values, row_ids, col_ids):
    flat_idx = row_ids * _C + col_ids
    pad = _NNZ_PAD - values.shape[0]
    idx = jnp.concatenate([flat_idx, jnp.zeros((pad,), jnp.int32)])
    val = jnp.concatenate([values, jnp.zeros((pad,), jnp.float32)])
    idx = idx.reshape(_NW, _CHUNK_ROWS, _LANES)
    val = val.reshape(_NW, _CHUNK_ROWS, _LANES)
    zero = jnp.zeros((_K_SIZE,), jnp.float32)
    mesh = plsc.VectorSubcoreMesh(core_axis_name="c", subcore_axis_name="s")
    fn = functools.partial(
        pl.kernel,
        mesh=mesh,
        out_type=jax.ShapeDtypeStruct((2, _K_SIZE), jnp.float32),
        scratch_types=[
            pltpu.VMEM((_CHUNK_ROWS, _LANES), jnp.int32),
            pltpu.VMEM((_CHUNK_ROWS, _LANES), jnp.float32),
            pltpu.VMEM_SHARED((_K_SIZE,), jnp.float32),
        ],
    )(_scatter_body)
    return fn(idx, val, zero)


def _mm_body(k_ref, x_ref, o_ref):
    kmat = (k_ref[0] + k_ref[1]).astype(jnp.bfloat16)
    o_ref[...] = jax.lax.dot_general(
        kmat, x_ref[...],
        dimension_numbers=(((1,), (0,)), ((), ())),
        preferred_element_type=jnp.float32,
    ).astype(jnp.bfloat16)


@functools.partial(jax.jit, static_argnames=("pb",))
def _matmul(kparts, x, pb=_PB):
    p = x.shape[1]
    return pl.pallas_call(
        _mm_body,
        grid=(p // pb,),
        in_specs=[
            pl.BlockSpec((2, _F, _C), lambda i: (0, 0, 0)),
            pl.BlockSpec((_C, pb), lambda i: (0, i)),
        ],
        out_specs=pl.BlockSpec((_F, pb), lambda i: (0, i)),
        out_shape=jax.ShapeDtypeStruct((_F, p), jnp.bfloat16),
        compiler_params=pltpu.CompilerParams(
            dimension_semantics=("parallel",),
        ),
    )(kparts, x)


def kernel(inputs, values, row_ids, col_ids):
    b, c, h, w = inputs.shape
    kparts = _build_kernel_coo(values, row_ids, col_ids).reshape(2, _F, _C)
    flat = inputs.reshape(c, h * w).astype(jnp.bfloat16)
    out = _matmul(kparts, flat)
    return out.astype(jnp.float32).reshape(b, _F, h, w)
